# Initial kernel scaffold; baseline (speedup 1.0000x reference)
import jax
import jax.numpy as jnp
from jax.experimental import pallas as pl


def kernel(rays_o, rays_d, grid, atoms, grid_id):
    raise NotImplementedError("write your pallas kernel here")


# speedup vs baseline: 4.0096x; 4.0096x over previous
import functools
import math

import jax
import jax.numpy as jnp
from jax import lax
from jax.experimental import pallas as pl
from jax.experimental.pallas import tpu as pltpu
from jax.experimental.pallas import tpu_sc as plsc

RADIUS_C = 1.3
COARSE_C = 64
FINE_C = 2
NUM_ATOMS_C = 32
DATA_DIM_C = 13
NB = 128
FINE_VOXEL = (RADIUS_C * 2.0 / COARSE_C) / FINE_C
STEP_C = FINE_VOXEL / 2.0
N_INT_C = int(math.sqrt(3.0) * RADIUS_C * 2.0 / STEP_C) - 1
NSAMP = N_INT_C - 1
NPAD = 448
C0_C = 0.28209479177387814
C1_C = 0.4886025119029199
GRES = COARSE_C * FINE_C
OFFS = [(-1, -1, -1), (-1, -1, 1), (-1, 1, -1), (-1, 1, 1),
        (1, -1, -1), (1, -1, 1), (1, 1, -1), (1, 1, 1)]

SC_NC = 2
SC_NS = 16
SC_NW = SC_NC * SC_NS
TOTAL_ROWS = NB * 8 * NPAD
ROWS_PER_W = TOTAL_ROWS // SC_NW
CHUNK = 128
NCHUNK = ROWS_PER_W // CHUNK


def _ray_geometry(o, d, iota_f):
    r = RADIUS_C
    start = None
    for dim in range(3):
        od = o[:, dim:dim + 1]
        dd = d[:, dim:dim + 1]
        off_pos = (r - od) / dd
        off_neg = (-r - od) / dd
        off_in = jnp.minimum(off_pos, off_neg)
        start = off_in if start is None else jnp.maximum(start, off_in)
    it = start + iota_f * STEP_C
    g = []
    inside = None
    for dim in range(3):
        pt = o[:, dim:dim + 1] + it * d[:, dim:dim + 1]
        ok = (pt > -r) & (pt < r)
        inside = ok if inside is None else (inside & ok)
        g.append((pt + r) / FINE_VOXEL)
    return g, inside


def _neighbor(g, k):
    w = None
    cidx = []
    fflat = None
    for dim in range(3):
        pre = g[dim] + OFFS[k][dim] * 0.5
        pf = jnp.clip(jnp.floor(pre), 0.0, GRES - 1.0)
        dist = jnp.abs(g[dim] - (pf + 0.5))
        wd = jnp.clip(1.0 - dist, 0.0, None)
        w = wd if w is None else (w * wd)
        ii = pf.astype(jnp.int32)
        cidx.append(ii >> 1)
        fb = ii & 1
        fflat = fb if fflat is None else (fflat * 2 + fb)
    cflat = (cidx[0] * COARSE_C + cidx[1]) * COARSE_C + cidx[2]
    return w, fflat, cflat


def _prep_body(o_ref, d_ref, cflat_ref):
    o = o_ref[...].reshape(NB, 3)
    d = d_ref[...].reshape(NB, 3)
    iota_i = lax.broadcasted_iota(jnp.int32, (NB, NPAD), 1)
    iota_f = iota_i.astype(jnp.float32)
    g, inside = _ray_geometry(o, d, iota_f)
    valid = inside & (iota_i < NSAMP)
    for k in range(8):
        _, _, cflat = _neighbor(g, k)
        cflat_ref[:, k, :] = jnp.where(valid, cflat, 0)


def _combine_body(rows_ref, o_ref, d_ref, a_ref, out_ref):
    o = o_ref[...].reshape(1, 3)
    d = d_ref[...].reshape(1, 3)
    iota_i = lax.broadcasted_iota(jnp.int32, (NPAD, 1), 0)
    iota_f = iota_i.astype(jnp.float32)
    g, inside = _ray_geometry(o, d, iota_f)
    valid = inside & (iota_i < NSAMP)
    rows = rows_ref[...].reshape(8, NPAD, NUM_ATOMS_C)
    wk = []
    fk = []
    for k in range(8):
        w, fflat, _ = _neighbor(g, k)
        wk.append(jnp.where(valid, w, 0.0))
        fk.append(fflat)
    vparts = []
    for fv in range(8):
        acc = None
        for k in range(8):
            sel = jnp.where(fk[k] == fv, wk[k], 0.0)
            term = sel * rows[k]
            acc = term if acc is None else (acc + term)
        vparts.append(acc)
    v = jnp.concatenate(vparts, axis=1)
    data = jnp.dot(v, a_ref[...], preferred_element_type=jnp.float32)
    x = d[:, 0:1]
    y = d[:, 1:2]
    z = d[:, 2:3]
    sigma = jnp.maximum(data[:, 12:13], 0.0)
    alpha = 1.0 - jnp.exp(-sigma * STEP_C)
    t = 1.0 - alpha + 1e-10
    s = 1
    while s < NPAD:
        shifted = jnp.concatenate(
            [jnp.ones((s, 1), jnp.float32), t[: NPAD - s]], axis=0)
        t = t * shifted
        s *= 2
    trans = jnp.concatenate(
        [jnp.ones((1, 1), jnp.float32), t[: NPAD - 1]], axis=0)
    wgt = alpha * trans
    outs = []
    for c in range(3):
        sc = (C0_C * data[:, 4 * c:4 * c + 1]
              - C1_C * y * data[:, 4 * c + 1:4 * c + 2]
              + C1_C * z * data[:, 4 * c + 2:4 * c + 3]
              - C1_C * x * data[:, 4 * c + 3:4 * c + 4])
        rgb = 1.0 / (1.0 + jnp.exp(-sc))
        outs.append(jnp.sum(wgt * rgb, axis=0, keepdims=True))
    outs.append(jnp.zeros((1, 1), jnp.float32))
    out_ref[...] = jnp.concatenate(outs, axis=1).reshape(1, 1, 4)


def _sc_gather_body(table_hbm, idx_hbm, out_hbm, idx_v, rows_v, sem):
    wid = lax.axis_index("s") * SC_NC + lax.axis_index("c")
    pltpu.sync_copy(idx_hbm.at[pl.ds(wid * NCHUNK, NCHUNK)], idx_v)

    def body(j, carry):
        pltpu.async_copy(table_hbm.at[idx_v.at[j]], rows_v, sem).wait()
        pltpu.sync_copy(rows_v,
                        out_hbm.at[pl.ds(wid * ROWS_PER_W + j * CHUNK, CHUNK)])
        return carry

    lax.fori_loop(0, NCHUNK, body, 0)


@functools.cache
def _sc_gather():
    return pl.kernel(
        _sc_gather_body,
        mesh=plsc.VectorSubcoreMesh(
            core_axis_name="c", subcore_axis_name="s", num_cores=SC_NC),
        out_type=jax.ShapeDtypeStruct((TOTAL_ROWS, NUM_ATOMS_C), jnp.float32),
        compiler_params=pltpu.CompilerParams(use_tc_tiling_on_sc=False),
        scratch_types=[
            pltpu.VMEM((NCHUNK, CHUNK), jnp.int32),
            pltpu.VMEM((CHUNK, NUM_ATOMS_C), jnp.float32),
            pltpu.SemaphoreType.DMA,
        ],
    )


def kernel(rays_o, rays_d, grid, atoms, grid_id):
    del grid_id
    o3 = rays_o.reshape(NB, 1, 3)
    d3 = rays_d.reshape(NB, 1, 3)

    cflat = pl.pallas_call(
        _prep_body,
        out_shape=jax.ShapeDtypeStruct((NB, 8, NPAD), jnp.int32),
        in_specs=[
            pl.BlockSpec((NB, 1, 3), lambda: (0, 0, 0)),
            pl.BlockSpec((NB, 1, 3), lambda: (0, 0, 0)),
        ],
        out_specs=pl.BlockSpec((NB, 8, NPAD), lambda: (0, 0, 0)),
    )(o3, d3)

    table = grid.reshape(COARSE_C * COARSE_C * COARSE_C, NUM_ATOMS_C)
    idx2d = cflat.reshape(TOTAL_ROWS // CHUNK, CHUNK)
    rows_flat = _sc_gather()(table, idx2d)
    rows4 = rows_flat.reshape(NB, 8, NPAD, NUM_ATOMS_C)

    a_flat = atoms.reshape(8 * NUM_ATOMS_C, DATA_DIM_C)
    a_pad = jnp.zeros((8 * NUM_ATOMS_C, 128), jnp.float32)
    a_pad = a_pad.at[:, :DATA_DIM_C].set(a_flat)

    out = pl.pallas_call(
        _combine_body,
        grid=(NB,),
        out_shape=jax.ShapeDtypeStruct((NB, 1, 4), jnp.float32),
        in_specs=[
            pl.BlockSpec((1, 8, NPAD, NUM_ATOMS_C), lambda b: (b, 0, 0, 0)),
            pl.BlockSpec((1, 1, 3), lambda b: (b, 0, 0)),
            pl.BlockSpec((1, 1, 3), lambda b: (b, 0, 0)),
            pl.BlockSpec((8 * NUM_ATOMS_C, 128), lambda b: (0, 0)),
        ],
        out_specs=pl.BlockSpec((1, 1, 4), lambda b: (b, 0, 0)),
    )(rows4, o3, d3, a_pad)

    return out.reshape(NB, 4)[:, :3]


# speedup vs baseline: 4.0201x; 1.0026x over previous
import functools
import math

import jax
import jax.numpy as jnp
from jax import lax
from jax.experimental import pallas as pl
from jax.experimental.pallas import tpu as pltpu
from jax.experimental.pallas import tpu_sc as plsc

RADIUS_C = 1.3
COARSE_C = 64
FINE_C = 2
NUM_ATOMS_C = 32
DATA_DIM_C = 13
NB = 128
FINE_VOXEL = (RADIUS_C * 2.0 / COARSE_C) / FINE_C
STEP_C = FINE_VOXEL / 2.0
N_INT_C = int(math.sqrt(3.0) * RADIUS_C * 2.0 / STEP_C) - 1
NSAMP = N_INT_C - 1
NPAD = 448
C0_C = 0.28209479177387814
C1_C = 0.4886025119029199
GRES = COARSE_C * FINE_C
OFFS = [(-1, -1, -1), (-1, -1, 1), (-1, 1, -1), (-1, 1, 1),
        (1, -1, -1), (1, -1, 1), (1, 1, -1), (1, 1, 1)]

SC_NC = 2
SC_NS = 16
SC_NW = SC_NC * SC_NS
TOTAL_ROWS = NB * 8 * NPAD
ROWS_PER_W = TOTAL_ROWS // SC_NW
CHUNK = 128
NCHUNK = ROWS_PER_W // CHUNK


def _ray_geometry(o, d, iota_f):
    r = RADIUS_C
    start = None
    for dim in range(3):
        od = o[:, dim:dim + 1]
        dd = d[:, dim:dim + 1]
        off_pos = (r - od) / dd
        off_neg = (-r - od) / dd
        off_in = jnp.minimum(off_pos, off_neg)
        start = off_in if start is None else jnp.maximum(start, off_in)
    it = start + iota_f * STEP_C
    g = []
    inside = None
    for dim in range(3):
        pt = o[:, dim:dim + 1] + it * d[:, dim:dim + 1]
        ok = (pt > -r) & (pt < r)
        inside = ok if inside is None else (inside & ok)
        g.append((pt + r) / FINE_VOXEL)
    return g, inside


def _neighbor(g, k):
    w = None
    cidx = []
    fflat = None
    for dim in range(3):
        pre = g[dim] + OFFS[k][dim] * 0.5
        pf = jnp.clip(jnp.floor(pre), 0.0, GRES - 1.0)
        dist = jnp.abs(g[dim] - (pf + 0.5))
        wd = jnp.clip(1.0 - dist, 0.0, None)
        w = wd if w is None else (w * wd)
        ii = pf.astype(jnp.int32)
        cidx.append(ii >> 1)
        fb = ii & 1
        fflat = fb if fflat is None else (fflat * 2 + fb)
    cflat = (cidx[0] * COARSE_C + cidx[1]) * COARSE_C + cidx[2]
    return w, fflat, cflat


def _prep_body(o_ref, d_ref, cflat_ref):
    o = o_ref[...].reshape(NB, 3)
    d = d_ref[...].reshape(NB, 3)
    iota_i = lax.broadcasted_iota(jnp.int32, (NB, NPAD), 1)
    iota_f = iota_i.astype(jnp.float32)
    g, inside = _ray_geometry(o, d, iota_f)
    valid = inside & (iota_i < NSAMP)
    for k in range(8):
        _, _, cflat = _neighbor(g, k)
        cflat_ref[:, k, :] = jnp.where(valid, cflat, 0)


def _combine_body(rows_ref, o_ref, d_ref, a_ref, out_ref):
    o = o_ref[...].reshape(1, 3)
    d = d_ref[...].reshape(1, 3)
    iota_i = lax.broadcasted_iota(jnp.int32, (NPAD, 1), 0)
    iota_f = iota_i.astype(jnp.float32)
    g, inside = _ray_geometry(o, d, iota_f)
    valid = inside & (iota_i < NSAMP)
    rows = rows_ref[...].reshape(8, NPAD, NUM_ATOMS_C)
    wk = []
    fk = []
    for k in range(8):
        w, fflat, _ = _neighbor(g, k)
        wk.append(jnp.where(valid, w, 0.0))
        fk.append(fflat)
    vparts = []
    for fv in range(8):
        acc = None
        for k in range(8):
            sel = jnp.where(fk[k] == fv, wk[k], 0.0)
            term = sel * rows[k]
            acc = term if acc is None else (acc + term)
        vparts.append(acc)
    v = jnp.concatenate(vparts, axis=1)
    data = jnp.dot(v, a_ref[...], preferred_element_type=jnp.float32)
    x = d[:, 0:1]
    y = d[:, 1:2]
    z = d[:, 2:3]
    sigma = jnp.maximum(data[:, 12:13], 0.0)
    alpha = 1.0 - jnp.exp(-sigma * STEP_C)
    t = 1.0 - alpha + 1e-10
    s = 1
    while s < NPAD:
        shifted = jnp.concatenate(
            [jnp.ones((s, 1), jnp.float32), t[: NPAD - s]], axis=0)
        t = t * shifted
        s *= 2
    trans = jnp.concatenate(
        [jnp.ones((1, 1), jnp.float32), t[: NPAD - 1]], axis=0)
    wgt = alpha * trans
    outs = []
    for c in range(3):
        sc = (C0_C * data[:, 4 * c:4 * c + 1]
              - C1_C * y * data[:, 4 * c + 1:4 * c + 2]
              + C1_C * z * data[:, 4 * c + 2:4 * c + 3]
              - C1_C * x * data[:, 4 * c + 3:4 * c + 4])
        rgb = 1.0 / (1.0 + jnp.exp(-sc))
        outs.append(jnp.sum(wgt * rgb, axis=0, keepdims=True))
    outs.append(jnp.zeros((1, 1), jnp.float32))
    out_ref[...] = jnp.concatenate(outs, axis=1).reshape(1, 1, 4)


SC_PIPE = 4


def _sc_gather_body(table_hbm, idx_hbm, out_hbm, idx_v, rows_v, semg, semw):
    wid = lax.axis_index("s") * SC_NC + lax.axis_index("c")
    pltpu.sync_copy(idx_hbm.at[pl.ds(wid * NCHUNK, NCHUNK)], idx_v)

    def body(g, carry):
        base_c = g * SC_PIPE
        cps = [
            pltpu.async_copy(
                table_hbm.at[idx_v.at[base_c + b]], rows_v.at[b], semg)
            for b in range(SC_PIPE)
        ]
        for cp in cps:
            cp.wait()
        wps = [
            pltpu.async_copy(
                rows_v.at[b],
                out_hbm.at[pl.ds(
                    wid * ROWS_PER_W + (base_c + b) * CHUNK, CHUNK)],
                semw)
            for b in range(SC_PIPE)
        ]
        for wp in wps:
            wp.wait()
        return carry

    lax.fori_loop(0, NCHUNK // SC_PIPE, body, 0)


@functools.cache
def _sc_gather():
    return pl.kernel(
        _sc_gather_body,
        mesh=plsc.VectorSubcoreMesh(
            core_axis_name="c", subcore_axis_name="s", num_cores=SC_NC),
        out_type=jax.ShapeDtypeStruct((TOTAL_ROWS, NUM_ATOMS_C), jnp.float32),
        compiler_params=pltpu.CompilerParams(use_tc_tiling_on_sc=False),
        scratch_types=[
            pltpu.VMEM((NCHUNK, CHUNK), jnp.int32),
            pltpu.VMEM((SC_PIPE, CHUNK, NUM_ATOMS_C), jnp.float32),
            pltpu.SemaphoreType.DMA,
            pltpu.SemaphoreType.DMA,
        ],
    )


def kernel(rays_o, rays_d, grid, atoms, grid_id):
    del grid_id
    o3 = rays_o.reshape(NB, 1, 3)
    d3 = rays_d.reshape(NB, 1, 3)

    cflat = pl.pallas_call(
        _prep_body,
        out_shape=jax.ShapeDtypeStruct((NB, 8, NPAD), jnp.int32),
        in_specs=[
            pl.BlockSpec((NB, 1, 3), lambda: (0, 0, 0)),
            pl.BlockSpec((NB, 1, 3), lambda: (0, 0, 0)),
        ],
        out_specs=pl.BlockSpec((NB, 8, NPAD), lambda: (0, 0, 0)),
    )(o3, d3)

    table = grid.reshape(COARSE_C * COARSE_C * COARSE_C, NUM_ATOMS_C)
    idx2d = cflat.reshape(TOTAL_ROWS // CHUNK, CHUNK)
    rows_flat = _sc_gather()(table, idx2d)
    rows4 = rows_flat.reshape(NB, 8, NPAD, NUM_ATOMS_C)

    a_flat = atoms.reshape(8 * NUM_ATOMS_C, DATA_DIM_C)
    a_pad = jnp.zeros((8 * NUM_ATOMS_C, 128), jnp.float32)
    a_pad = a_pad.at[:, :DATA_DIM_C].set(a_flat)

    out = pl.pallas_call(
        _combine_body,
        grid=(NB,),
        out_shape=jax.ShapeDtypeStruct((NB, 1, 4), jnp.float32),
        in_specs=[
            pl.BlockSpec((1, 8, NPAD, NUM_ATOMS_C), lambda b: (b, 0, 0, 0)),
            pl.BlockSpec((1, 1, 3), lambda b: (b, 0, 0)),
            pl.BlockSpec((1, 1, 3), lambda b: (b, 0, 0)),
            pl.BlockSpec((8 * NUM_ATOMS_C, 128), lambda b: (0, 0)),
        ],
        out_specs=pl.BlockSpec((1, 1, 4), lambda b: (b, 0, 0)),
    )(rows4, o3, d3, a_pad)

    return out.reshape(NB, 4)[:, :3]


# speedup vs baseline: 4.0228x; 1.0007x over previous
import functools
import math

import jax
import jax.numpy as jnp
from jax import lax
from jax.experimental import pallas as pl
from jax.experimental.pallas import tpu as pltpu
from jax.experimental.pallas import tpu_sc as plsc

RADIUS_C = 1.3
COARSE_C = 64
FINE_C = 2
NUM_ATOMS_C = 32
DATA_DIM_C = 13
NB = 128
FINE_VOXEL = (RADIUS_C * 2.0 / COARSE_C) / FINE_C
STEP_C = FINE_VOXEL / 2.0
N_INT_C = int(math.sqrt(3.0) * RADIUS_C * 2.0 / STEP_C) - 1
NSAMP = N_INT_C - 1
NPAD = 448
C0_C = 0.28209479177387814
C1_C = 0.4886025119029199
GRES = COARSE_C * FINE_C
OFFS = [(-1, -1, -1), (-1, -1, 1), (-1, 1, -1), (-1, 1, 1),
        (1, -1, -1), (1, -1, 1), (1, 1, -1), (1, 1, 1)]

SC_NC = 2
SC_NS = 16
SC_NW = SC_NC * SC_NS
TOTAL_ROWS = NB * 8 * NPAD
ROWS_PER_W = TOTAL_ROWS // SC_NW
CHUNK = 512
NCHUNK = ROWS_PER_W // CHUNK


def _ray_geometry(o, d, iota_f):
    r = RADIUS_C
    start = None
    for dim in range(3):
        od = o[:, dim:dim + 1]
        dd = d[:, dim:dim + 1]
        off_pos = (r - od) / dd
        off_neg = (-r - od) / dd
        off_in = jnp.minimum(off_pos, off_neg)
        start = off_in if start is None else jnp.maximum(start, off_in)
    it = start + iota_f * STEP_C
    g = []
    inside = None
    for dim in range(3):
        pt = o[:, dim:dim + 1] + it * d[:, dim:dim + 1]
        ok = (pt > -r) & (pt < r)
        inside = ok if inside is None else (inside & ok)
        g.append((pt + r) / FINE_VOXEL)
    return g, inside


def _neighbor(g, k):
    w = None
    cidx = []
    fflat = None
    for dim in range(3):
        pre = g[dim] + OFFS[k][dim] * 0.5
        pf = jnp.clip(jnp.floor(pre), 0.0, GRES - 1.0)
        dist = jnp.abs(g[dim] - (pf + 0.5))
        wd = jnp.clip(1.0 - dist, 0.0, None)
        w = wd if w is None else (w * wd)
        ii = pf.astype(jnp.int32)
        cidx.append(ii >> 1)
        fb = ii & 1
        fflat = fb if fflat is None else (fflat * 2 + fb)
    cflat = (cidx[0] * COARSE_C + cidx[1]) * COARSE_C + cidx[2]
    return w, fflat, cflat


def _prep_body(o_ref, d_ref, cflat_ref):
    o = o_ref[...].reshape(NB, 3)
    d = d_ref[...].reshape(NB, 3)
    iota_i = lax.broadcasted_iota(jnp.int32, (NB, NPAD), 1)
    iota_f = iota_i.astype(jnp.float32)
    g, inside = _ray_geometry(o, d, iota_f)
    valid = inside & (iota_i < NSAMP)
    for k in range(8):
        _, _, cflat = _neighbor(g, k)
        cflat_ref[:, k, :] = jnp.where(valid, cflat, 0)


def _combine_body(rows_ref, o_ref, d_ref, a_ref, out_ref):
    o = o_ref[...].reshape(1, 3)
    d = d_ref[...].reshape(1, 3)
    iota_i = lax.broadcasted_iota(jnp.int32, (NPAD, 1), 0)
    iota_f = iota_i.astype(jnp.float32)
    g, inside = _ray_geometry(o, d, iota_f)
    valid = inside & (iota_i < NSAMP)
    rows = rows_ref[...].reshape(8, NPAD, NUM_ATOMS_C)
    wk = []
    fk = []
    for k in range(8):
        w, fflat, _ = _neighbor(g, k)
        wk.append(jnp.where(valid, w, 0.0))
        fk.append(fflat)
    vparts = []
    for fv in range(8):
        acc = None
        for k in range(8):
            sel = jnp.where(fk[k] == fv, wk[k], 0.0)
            term = sel * rows[k]
            acc = term if acc is None else (acc + term)
        vparts.append(acc)
    v = jnp.concatenate(vparts, axis=1)
    data = jnp.dot(v, a_ref[...], preferred_element_type=jnp.float32)
    x = d[:, 0:1]
    y = d[:, 1:2]
    z = d[:, 2:3]
    sigma = jnp.maximum(data[:, 12:13], 0.0)
    alpha = 1.0 - jnp.exp(-sigma * STEP_C)
    t = 1.0 - alpha + 1e-10
    s = 1
    while s < NPAD:
        shifted = jnp.concatenate(
            [jnp.ones((s, 1), jnp.float32), t[: NPAD - s]], axis=0)
        t = t * shifted
        s *= 2
    trans = jnp.concatenate(
        [jnp.ones((1, 1), jnp.float32), t[: NPAD - 1]], axis=0)
    wgt = alpha * trans
    outs = []
    for c in range(3):
        sc = (C0_C * data[:, 4 * c:4 * c + 1]
              - C1_C * y * data[:, 4 * c + 1:4 * c + 2]
              + C1_C * z * data[:, 4 * c + 2:4 * c + 3]
              - C1_C * x * data[:, 4 * c + 3:4 * c + 4])
        rgb = 1.0 / (1.0 + jnp.exp(-sc))
        outs.append(jnp.sum(wgt * rgb, axis=0, keepdims=True))
    outs.append(jnp.zeros((1, 1), jnp.float32))
    out_ref[...] = jnp.concatenate(outs, axis=1).reshape(1, 1, 4)


SC_PIPE = 4


def _sc_gather_body(table_hbm, idx_hbm, out_hbm, idx_v, rows_v, semg, semw):
    wid = lax.axis_index("s") * SC_NC + lax.axis_index("c")
    pltpu.sync_copy(idx_hbm.at[pl.ds(wid * NCHUNK, NCHUNK)], idx_v)

    def body(g, carry):
        base_c = g * SC_PIPE
        cps = [
            pltpu.async_copy(
                table_hbm.at[idx_v.at[base_c + b]], rows_v.at[b], semg)
            for b in range(SC_PIPE)
        ]
        for cp in cps:
            cp.wait()
        wps = [
            pltpu.async_copy(
                rows_v.at[b],
                out_hbm.at[pl.ds(
                    wid * ROWS_PER_W + (base_c + b) * CHUNK, CHUNK)],
                semw)
            for b in range(SC_PIPE)
        ]
        for wp in wps:
            wp.wait()
        return carry

    lax.fori_loop(0, NCHUNK // SC_PIPE, body, 0)


@functools.cache
def _sc_gather():
    return pl.kernel(
        _sc_gather_body,
        mesh=plsc.VectorSubcoreMesh(
            core_axis_name="c", subcore_axis_name="s", num_cores=SC_NC),
        out_type=jax.ShapeDtypeStruct((TOTAL_ROWS, NUM_ATOMS_C), jnp.float32),
        compiler_params=pltpu.CompilerParams(use_tc_tiling_on_sc=False),
        scratch_types=[
            pltpu.VMEM((NCHUNK, CHUNK), jnp.int32),
            pltpu.VMEM((SC_PIPE, CHUNK, NUM_ATOMS_C), jnp.float32),
            pltpu.SemaphoreType.DMA,
            pltpu.SemaphoreType.DMA,
        ],
    )


def kernel(rays_o, rays_d, grid, atoms, grid_id):
    del grid_id
    o3 = rays_o.reshape(NB, 1, 3)
    d3 = rays_d.reshape(NB, 1, 3)

    cflat = pl.pallas_call(
        _prep_body,
        out_shape=jax.ShapeDtypeStruct((NB, 8, NPAD), jnp.int32),
        in_specs=[
            pl.BlockSpec((NB, 1, 3), lambda: (0, 0, 0)),
            pl.BlockSpec((NB, 1, 3), lambda: (0, 0, 0)),
        ],
        out_specs=pl.BlockSpec((NB, 8, NPAD), lambda: (0, 0, 0)),
    )(o3, d3)

    table = grid.reshape(COARSE_C * COARSE_C * COARSE_C, NUM_ATOMS_C)
    idx2d = cflat.reshape(TOTAL_ROWS // CHUNK, CHUNK)
    rows_flat = _sc_gather()(table, idx2d)
    rows4 = rows_flat.reshape(NB, 8, NPAD, NUM_ATOMS_C)

    a_flat = atoms.reshape(8 * NUM_ATOMS_C, DATA_DIM_C)
    a_pad = jnp.zeros((8 * NUM_ATOMS_C, 128), jnp.float32)
    a_pad = a_pad.at[:, :DATA_DIM_C].set(a_flat)

    out = pl.pallas_call(
        _combine_body,
        grid=(NB,),
        out_shape=jax.ShapeDtypeStruct((NB, 1, 4), jnp.float32),
        in_specs=[
            pl.BlockSpec((1, 8, NPAD, NUM_ATOMS_C), lambda b: (b, 0, 0, 0)),
            pl.BlockSpec((1, 1, 3), lambda b: (b, 0, 0)),
            pl.BlockSpec((1, 1, 3), lambda b: (b, 0, 0)),
            pl.BlockSpec((8 * NUM_ATOMS_C, 128), lambda b: (0, 0)),
        ],
        out_specs=pl.BlockSpec((1, 1, 4), lambda b: (b, 0, 0)),
    )(rows4, o3, d3, a_pad)

    return out.reshape(NB, 4)[:, :3]


# speedup vs baseline: 5.7004x; 1.4170x over previous
import functools
import math

import jax
import jax.numpy as jnp
from jax import lax
from jax.experimental import pallas as pl
from jax.experimental.pallas import tpu as pltpu
from jax.experimental.pallas import tpu_sc as plsc

RADIUS_C = 1.3
COARSE_C = 64
FINE_C = 2
NUM_ATOMS_C = 32
DATA_DIM_C = 13
NB = 128
FINE_VOXEL = (RADIUS_C * 2.0 / COARSE_C) / FINE_C
STEP_C = FINE_VOXEL / 2.0
N_INT_C = int(math.sqrt(3.0) * RADIUS_C * 2.0 / STEP_C) - 1
NSAMP = N_INT_C - 1
NPAD = 448
C0_C = 0.28209479177387814
C1_C = 0.4886025119029199
GRES = COARSE_C * FINE_C
OFFS = [(-1, -1, -1), (-1, -1, 1), (-1, 1, -1), (-1, 1, 1),
        (1, -1, -1), (1, -1, 1), (1, 1, -1), (1, 1, 1)]

SC_NC = 2
SC_NS = 16
SC_NW = SC_NC * SC_NS
TOTAL_ROWS = NB * 8 * NPAD
ROWS_PER_W = TOTAL_ROWS // SC_NW
CHUNK = 512
NCHUNK = ROWS_PER_W // CHUNK


def _ray_geometry(o, d, iota_f):
    r = RADIUS_C
    start = None
    for dim in range(3):
        od = o[:, dim:dim + 1]
        dd = d[:, dim:dim + 1]
        off_pos = (r - od) / dd
        off_neg = (-r - od) / dd
        off_in = jnp.minimum(off_pos, off_neg)
        start = off_in if start is None else jnp.maximum(start, off_in)
    it = start + iota_f * STEP_C
    g = []
    inside = None
    for dim in range(3):
        pt = o[:, dim:dim + 1] + it * d[:, dim:dim + 1]
        ok = (pt > -r) & (pt < r)
        inside = ok if inside is None else (inside & ok)
        g.append((pt + r) / FINE_VOXEL)
    return g, inside


def _neighbor(g, k):
    w = None
    cidx = []
    fflat = None
    for dim in range(3):
        pre = g[dim] + OFFS[k][dim] * 0.5
        pf = jnp.clip(jnp.floor(pre), 0.0, GRES - 1.0)
        dist = jnp.abs(g[dim] - (pf + 0.5))
        wd = jnp.clip(1.0 - dist, 0.0, None)
        w = wd if w is None else (w * wd)
        ii = pf.astype(jnp.int32)
        cidx.append(ii >> 1)
        fb = ii & 1
        fflat = fb if fflat is None else (fflat * 2 + fb)
    cflat = (cidx[0] * COARSE_C + cidx[1]) * COARSE_C + cidx[2]
    return w, fflat, cflat


def _prep_body(o_ref, d_ref, cflat_ref):
    o = o_ref[...].reshape(NB, 3)
    d = d_ref[...].reshape(NB, 3)
    iota_i = lax.broadcasted_iota(jnp.int32, (NB, NPAD), 1)
    iota_f = iota_i.astype(jnp.float32)
    g, inside = _ray_geometry(o, d, iota_f)
    valid = inside & (iota_i < NSAMP)
    for k in range(8):
        _, _, cflat = _neighbor(g, k)
        cflat_ref[:, k, :] = jnp.where(valid, cflat, 0)


RPB = 8


def _combine_body(rows_ref, oT_ref, dT_ref, a_ref, out_ref):
    oT = oT_ref[...]
    dT = dT_ref[...]
    o = [oT[:, dim, :] for dim in range(3)]
    d = [dT[:, dim, :] for dim in range(3)]
    iota_i = lax.broadcasted_iota(jnp.int32, (NPAD, RPB), 0)
    iota_f = iota_i.astype(jnp.float32)
    r = RADIUS_C
    start = None
    for dim in range(3):
        off_pos = (r - o[dim]) / d[dim]
        off_neg = (-r - o[dim]) / d[dim]
        off_in = jnp.minimum(off_pos, off_neg)
        start = off_in if start is None else jnp.maximum(start, off_in)
    it = start + iota_f * STEP_C
    g = []
    inside = None
    for dim in range(3):
        pt = o[dim] + it * d[dim]
        ok = (pt > -r) & (pt < r)
        inside = ok if inside is None else (inside & ok)
        g.append((pt + r) / FINE_VOXEL)
    valid = inside & (iota_i < NSAMP)
    wk = []
    fk = []
    for k in range(8):
        w, fflat, _ = _neighbor(g, k)
        wk.append(jnp.where(valid, w, 0.0))
        fk.append(fflat)
    vs = []
    for ri in range(RPB):
        vparts = []
        for fv in range(8):
            acc = None
            for k in range(8):
                sel = jnp.where(fk[k][:, ri:ri + 1] == fv,
                                wk[k][:, ri:ri + 1], 0.0)
                term = sel * rows_ref[ri, k].astype(jnp.float32)
                acc = term if acc is None else (acc + term)
            vparts.append(acc)
        vs.append(jnp.concatenate(vparts, axis=1))
    v = jnp.concatenate(vs, axis=0)
    data = jnp.dot(v, a_ref[...], preferred_element_type=jnp.float32)
    x = dT[:, 0, :]
    y = dT[:, 1, :]
    z = dT[:, 2, :]
    sig_cols = []
    s_cols = [[], [], []]
    for ri in range(RPB):
        dr = data[ri * NPAD:(ri + 1) * NPAD]
        sig_cols.append(dr[:, 12:13])
        for c in range(3):
            sc = (C0_C * dr[:, 4 * c:4 * c + 1]
                  - C1_C * y[:, ri:ri + 1] * dr[:, 4 * c + 1:4 * c + 2]
                  + C1_C * z[:, ri:ri + 1] * dr[:, 4 * c + 2:4 * c + 3]
                  - C1_C * x[:, ri:ri + 1] * dr[:, 4 * c + 3:4 * c + 4])
            s_cols[c].append(sc)
    sigma = jnp.maximum(jnp.concatenate(sig_cols, axis=1), 0.0)
    alpha = 1.0 - jnp.exp(-sigma * STEP_C)
    t = 1.0 - alpha + 1e-10
    s = 1
    while s < NPAD:
        shifted = jnp.concatenate(
            [jnp.ones((s, RPB), jnp.float32), t[: NPAD - s]], axis=0)
        t = t * shifted
        s *= 2
    trans = jnp.concatenate(
        [jnp.ones((1, RPB), jnp.float32), t[: NPAD - 1]], axis=0)
    wgt = alpha * trans
    outs = []
    for c in range(3):
        sc = jnp.concatenate(s_cols[c], axis=1)
        rgb = 1.0 / (1.0 + jnp.exp(-sc))
        outs.append(jnp.sum(wgt * rgb, axis=0, keepdims=True))
    outs.append(jnp.zeros((1, RPB), jnp.float32))
    out_ref[...] = jnp.concatenate(outs, axis=0).reshape(1, 4, RPB)


SC_PIPE = 4


def _sc_gather_body(table_hbm, idx_hbm, out_hbm, idx_v, rows_v, semg, semw):
    wid = lax.axis_index("s") * SC_NC + lax.axis_index("c")
    pltpu.sync_copy(idx_hbm.at[pl.ds(wid * NCHUNK, NCHUNK)], idx_v)

    def body(g, carry):
        base_c = g * SC_PIPE
        cps = [
            pltpu.async_copy(
                table_hbm.at[idx_v.at[base_c + b]], rows_v.at[b], semg)
            for b in range(SC_PIPE)
        ]
        for cp in cps:
            cp.wait()
        wps = [
            pltpu.async_copy(
                rows_v.at[b],
                out_hbm.at[pl.ds(
                    wid * ROWS_PER_W + (base_c + b) * CHUNK, CHUNK)],
                semw)
            for b in range(SC_PIPE)
        ]
        for wp in wps:
            wp.wait()
        return carry

    lax.fori_loop(0, NCHUNK // SC_PIPE, body, 0)


@functools.cache
def _sc_gather():
    return pl.kernel(
        _sc_gather_body,
        mesh=plsc.VectorSubcoreMesh(
            core_axis_name="c", subcore_axis_name="s", num_cores=SC_NC),
        out_type=jax.ShapeDtypeStruct((TOTAL_ROWS, NUM_ATOMS_C), jnp.bfloat16),
        compiler_params=pltpu.CompilerParams(use_tc_tiling_on_sc=False),
        scratch_types=[
            pltpu.VMEM((NCHUNK, CHUNK), jnp.int32),
            pltpu.VMEM((SC_PIPE, CHUNK, NUM_ATOMS_C), jnp.bfloat16),
            pltpu.SemaphoreType.DMA,
            pltpu.SemaphoreType.DMA,
        ],
    )


def kernel(rays_o, rays_d, grid, atoms, grid_id):
    del grid_id
    o3 = rays_o.reshape(NB, 1, 3)
    d3 = rays_d.reshape(NB, 1, 3)

    cflat = pl.pallas_call(
        _prep_body,
        out_shape=jax.ShapeDtypeStruct((NB, 8, NPAD), jnp.int32),
        in_specs=[
            pl.BlockSpec((NB, 1, 3), lambda: (0, 0, 0)),
            pl.BlockSpec((NB, 1, 3), lambda: (0, 0, 0)),
        ],
        out_specs=pl.BlockSpec((NB, 8, NPAD), lambda: (0, 0, 0)),
    )(o3, d3)

    table = grid.reshape(COARSE_C * COARSE_C * COARSE_C,
                         NUM_ATOMS_C).astype(jnp.bfloat16)
    idx2d = cflat.reshape(TOTAL_ROWS // CHUNK, CHUNK)
    rows_flat = _sc_gather()(table, idx2d)
    rows4 = rows_flat.reshape(NB, 8, NPAD, NUM_ATOMS_C)

    a_flat = atoms.reshape(8 * NUM_ATOMS_C, DATA_DIM_C)
    a_pad = jnp.zeros((8 * NUM_ATOMS_C, 128), jnp.float32)
    a_pad = a_pad.at[:, :DATA_DIM_C].set(a_flat)

    oT = rays_o.reshape(NB // RPB, RPB, 3).transpose(0, 2, 1)
    dT = rays_d.reshape(NB // RPB, RPB, 3).transpose(0, 2, 1)
    out = pl.pallas_call(
        _combine_body,
        grid=(NB // RPB,),
        out_shape=jax.ShapeDtypeStruct((NB // RPB, 4, RPB), jnp.float32),
        in_specs=[
            pl.BlockSpec((RPB, 8, NPAD, NUM_ATOMS_C),
                         lambda b: (b, 0, 0, 0)),
            pl.BlockSpec((1, 3, RPB), lambda b: (b, 0, 0)),
            pl.BlockSpec((1, 3, RPB), lambda b: (b, 0, 0)),
            pl.BlockSpec((8 * NUM_ATOMS_C, 128), lambda b: (0, 0)),
        ],
        out_specs=pl.BlockSpec((1, 4, RPB), lambda b: (b, 0, 0)),
    )(rows4, oT, dT, a_pad)

    return out.transpose(0, 2, 1).reshape(NB, 4)[:, :3]


# speedup vs baseline: 7.5158x; 1.3185x over previous
import functools
import math

import jax
import jax.numpy as jnp
from jax import lax
from jax.experimental import pallas as pl
from jax.experimental.pallas import tpu as pltpu
from jax.experimental.pallas import tpu_sc as plsc

RADIUS_C = 1.3
COARSE_C = 64
FINE_C = 2
NUM_ATOMS_C = 32
DATA_DIM_C = 13
NB = 128
FINE_VOXEL = (RADIUS_C * 2.0 / COARSE_C) / FINE_C
STEP_C = FINE_VOXEL / 2.0
N_INT_C = int(math.sqrt(3.0) * RADIUS_C * 2.0 / STEP_C) - 1
NSAMP = N_INT_C - 1
NPAD = 448
C0_C = 0.28209479177387814
C1_C = 0.4886025119029199
GRES = COARSE_C * FINE_C
OFFS = [(-1, -1, -1), (-1, -1, 1), (-1, 1, -1), (-1, 1, 1),
        (1, -1, -1), (1, -1, 1), (1, 1, -1), (1, 1, 1)]

SC_NC = 2
SC_NS = 16
SC_NW = SC_NC * SC_NS
TOTAL_ROWS = NB * 8 * NPAD
ROWS_PER_W = TOTAL_ROWS // SC_NW
CHUNK = 512
NCHUNK = ROWS_PER_W // CHUNK


def _ray_geometry(o, d, iota_f):
    r = RADIUS_C
    start = None
    for dim in range(3):
        od = o[:, dim:dim + 1]
        dd = d[:, dim:dim + 1]
        off_pos = (r - od) / dd
        off_neg = (-r - od) / dd
        off_in = jnp.minimum(off_pos, off_neg)
        start = off_in if start is None else jnp.maximum(start, off_in)
    it = start + iota_f * STEP_C
    g = []
    inside = None
    for dim in range(3):
        pt = o[:, dim:dim + 1] + it * d[:, dim:dim + 1]
        ok = (pt > -r) & (pt < r)
        inside = ok if inside is None else (inside & ok)
        g.append((pt + r) / FINE_VOXEL)
    return g, inside


def _neighbor(g, k):
    w = None
    cidx = []
    fflat = None
    for dim in range(3):
        pre = g[dim] + OFFS[k][dim] * 0.5
        pf = jnp.clip(jnp.floor(pre), 0.0, GRES - 1.0)
        dist = jnp.abs(g[dim] - (pf + 0.5))
        wd = jnp.clip(1.0 - dist, 0.0, None)
        w = wd if w is None else (w * wd)
        ii = pf.astype(jnp.int32)
        cidx.append(ii >> 1)
        fb = ii & 1
        fflat = fb if fflat is None else (fflat * 2 + fb)
    cflat = (cidx[0] * COARSE_C + cidx[1]) * COARSE_C + cidx[2]
    return w, fflat, cflat


def _prep_body(o_ref, d_ref, cflat_ref):
    o = o_ref[...].reshape(NB, 3)
    d = d_ref[...].reshape(NB, 3)
    iota_i = lax.broadcasted_iota(jnp.int32, (NB, NPAD), 1)
    iota_f = iota_i.astype(jnp.float32)
    g, inside = _ray_geometry(o, d, iota_f)
    valid = inside & (iota_i < NSAMP)
    for k in range(8):
        _, _, cflat = _neighbor(g, k)
        cflat_ref[:, k, :] = jnp.where(valid, cflat, 0)


RPB = 8


def _combine_body(rows_ref, oT_ref, dT_ref, a_ref, out_ref):
    oT = oT_ref[...]
    dT = dT_ref[...]
    o = [oT[:, dim, :] for dim in range(3)]
    d = [dT[:, dim, :] for dim in range(3)]
    iota_i = lax.broadcasted_iota(jnp.int32, (NPAD, RPB), 0)
    iota_f = iota_i.astype(jnp.float32)
    r = RADIUS_C
    start = None
    for dim in range(3):
        off_pos = (r - o[dim]) / d[dim]
        off_neg = (-r - o[dim]) / d[dim]
        off_in = jnp.minimum(off_pos, off_neg)
        start = off_in if start is None else jnp.maximum(start, off_in)
    it = start + iota_f * STEP_C
    g = []
    inside = None
    for dim in range(3):
        pt = o[dim] + it * d[dim]
        ok = (pt > -r) & (pt < r)
        inside = ok if inside is None else (inside & ok)
        g.append((pt + r) / FINE_VOXEL)
    valid = inside & (iota_i < NSAMP)
    wk = []
    fk = []
    for k in range(8):
        w, fflat, _ = _neighbor(g, k)
        wk.append(jnp.where(valid, w, 0.0))
        fk.append(fflat)
    fcol16 = lax.broadcasted_iota(jnp.int32, (NPAD, 128), 1) // 16
    x = dT[:, 0, :]
    y = dT[:, 1, :]
    z = dT[:, 2, :]
    sig_cols = []
    s_cols = [[], [], []]
    for ri in range(RPB):
        p = jnp.dot(rows_ref[ri], a_ref[...],
                    preferred_element_type=jnp.float32)
        dr = None
        for k in range(8):
            slab = p[:, k * 128:(k + 1) * 128]
            sel = jnp.where(fk[k][:, ri:ri + 1] == fcol16, slab, 0.0)
            t = sel[:, :64] + sel[:, 64:]
            t = t[:, :32] + t[:, 32:]
            t = t[:, :16] + t[:, 16:]
            term = wk[k][:, ri:ri + 1] * t
            dr = term if dr is None else (dr + term)
        sig_cols.append(dr[:, 12:13])
        for c in range(3):
            sc = (C0_C * dr[:, 4 * c:4 * c + 1]
                  - C1_C * y[:, ri:ri + 1] * dr[:, 4 * c + 1:4 * c + 2]
                  + C1_C * z[:, ri:ri + 1] * dr[:, 4 * c + 2:4 * c + 3]
                  - C1_C * x[:, ri:ri + 1] * dr[:, 4 * c + 3:4 * c + 4])
            s_cols[c].append(sc)
    sigma = jnp.maximum(jnp.concatenate(sig_cols, axis=1), 0.0)
    alpha = 1.0 - jnp.exp(-sigma * STEP_C)
    t = 1.0 - alpha + 1e-10
    s = 1
    while s < NPAD:
        shifted = jnp.concatenate(
            [jnp.ones((s, RPB), jnp.float32), t[: NPAD - s]], axis=0)
        t = t * shifted
        s *= 2
    trans = jnp.concatenate(
        [jnp.ones((1, RPB), jnp.float32), t[: NPAD - 1]], axis=0)
    wgt = alpha * trans
    outs = []
    for c in range(3):
        sc = jnp.concatenate(s_cols[c], axis=1)
        rgb = 1.0 / (1.0 + jnp.exp(-sc))
        outs.append(jnp.sum(wgt * rgb, axis=0, keepdims=True))
    outs.append(jnp.zeros((1, RPB), jnp.float32))
    out_ref[...] = jnp.concatenate(outs, axis=0).reshape(1, 4, RPB)


SC_PIPE = 4


def _sc_gather_body(table_hbm, idx_hbm, out_hbm, idx_v, rows_v, semg, semw):
    wid = lax.axis_index("s") * SC_NC + lax.axis_index("c")
    pltpu.sync_copy(idx_hbm.at[pl.ds(wid * NCHUNK, NCHUNK)], idx_v)

    def body(g, carry):
        base_c = g * SC_PIPE
        cps = [
            pltpu.async_copy(
                table_hbm.at[idx_v.at[base_c + b]], rows_v.at[b], semg)
            for b in range(SC_PIPE)
        ]
        for cp in cps:
            cp.wait()
        wps = [
            pltpu.async_copy(
                rows_v.at[b],
                out_hbm.at[pl.ds(
                    wid * ROWS_PER_W + (base_c + b) * CHUNK, CHUNK)],
                semw)
            for b in range(SC_PIPE)
        ]
        for wp in wps:
            wp.wait()
        return carry

    lax.fori_loop(0, NCHUNK // SC_PIPE, body, 0)


@functools.cache
def _sc_gather():
    return pl.kernel(
        _sc_gather_body,
        mesh=plsc.VectorSubcoreMesh(
            core_axis_name="c", subcore_axis_name="s", num_cores=SC_NC),
        out_type=jax.ShapeDtypeStruct((TOTAL_ROWS, NUM_ATOMS_C), jnp.bfloat16),
        compiler_params=pltpu.CompilerParams(use_tc_tiling_on_sc=False),
        scratch_types=[
            pltpu.VMEM((NCHUNK, CHUNK), jnp.int32),
            pltpu.VMEM((SC_PIPE, CHUNK, NUM_ATOMS_C), jnp.bfloat16),
            pltpu.SemaphoreType.DMA,
            pltpu.SemaphoreType.DMA,
        ],
    )


def kernel(rays_o, rays_d, grid, atoms, grid_id):
    del grid_id
    o3 = rays_o.reshape(NB, 1, 3)
    d3 = rays_d.reshape(NB, 1, 3)

    cflat = pl.pallas_call(
        _prep_body,
        out_shape=jax.ShapeDtypeStruct((NB, 8, NPAD), jnp.int32),
        in_specs=[
            pl.BlockSpec((NB, 1, 3), lambda: (0, 0, 0)),
            pl.BlockSpec((NB, 1, 3), lambda: (0, 0, 0)),
        ],
        out_specs=pl.BlockSpec((NB, 8, NPAD), lambda: (0, 0, 0)),
    )(o3, d3)

    table = grid.reshape(COARSE_C * COARSE_C * COARSE_C,
                         NUM_ATOMS_C).astype(jnp.bfloat16)
    idx2d = cflat.transpose(0, 2, 1).reshape(TOTAL_ROWS // CHUNK, CHUNK)
    rows_flat = _sc_gather()(table, idx2d)
    rows4 = rows_flat.reshape(NB, NPAD, 8 * NUM_ATOMS_C)

    a_flat = atoms.reshape(8, NUM_ATOMS_C, DATA_DIM_C).transpose(1, 0, 2)
    a_all = jnp.zeros((NUM_ATOMS_C, 8, 16), jnp.float32)
    a_all = a_all.at[:, :, :DATA_DIM_C].set(a_flat).reshape(NUM_ATOMS_C, 128)
    eye8 = jnp.eye(8, dtype=jnp.float32)
    a_big = (eye8[:, None, :, None] * a_all[None, :, None, :]).reshape(
        8 * NUM_ATOMS_C, 8 * 128).astype(jnp.bfloat16)

    oT = rays_o.reshape(NB // RPB, RPB, 3).transpose(0, 2, 1)
    dT = rays_d.reshape(NB // RPB, RPB, 3).transpose(0, 2, 1)
    out = pl.pallas_call(
        _combine_body,
        grid=(NB // RPB,),
        out_shape=jax.ShapeDtypeStruct((NB // RPB, 4, RPB), jnp.float32),
        in_specs=[
            pl.BlockSpec((RPB, NPAD, 8 * NUM_ATOMS_C),
                         lambda b: (b, 0, 0)),
            pl.BlockSpec((1, 3, RPB), lambda b: (b, 0, 0)),
            pl.BlockSpec((1, 3, RPB), lambda b: (b, 0, 0)),
            pl.BlockSpec((8 * NUM_ATOMS_C, 8 * 128), lambda b: (0, 0)),
        ],
        out_specs=pl.BlockSpec((1, 4, RPB), lambda b: (b, 0, 0)),
    )(rows4, oT, dT, a_big)

    return out.transpose(0, 2, 1).reshape(NB, 4)[:, :3]
